# baseline (speedup 1.0000x reference)
import functools

import jax
import jax.numpy as jnp
from jax import lax
from jax.experimental import pallas as pl
from jax.experimental.pallas import tpu as pltpu
from jax.experimental.pallas import tpu_sc as plsc

N_NODES = 10000
D_FEAT = 128
NUM_GRAPHS = 256
GLOBAL_DIM = 64
HIDDEN_DIM = 256

CHUNK = 80
N_CHUNKS = N_NODES // CHUNK
N_WORKERS = 32
FULL_WORKERS = N_CHUNKS - 3 * N_WORKERS
PAD_N = 10240


def _sc_segment_sums(x, batch_i32, zeros_c):
    mesh = plsc.VectorSubcoreMesh(core_axis_name="c", subcore_axis_name="s")

    @functools.partial(
        pl.kernel,
        out_type=jax.ShapeDtypeStruct((2 * NUM_GRAPHS, D_FEAT), jnp.float32),
        mesh=mesh,
        scratch_types=(
            [pltpu.VMEM((CHUNK,), jnp.int32) for _ in range(4)]
            + [pltpu.VMEM((CHUNK, D_FEAT), jnp.float32) for _ in range(4)]
            + [pltpu.SemaphoreType.DMA for _ in range(8)]
            + [pltpu.VMEM_SHARED((NUM_GRAPHS, D_FEAT), jnp.float32)]
        ),
    )
    def seg(x_hbm, b_hbm, z_hbm, sums_out,
            i0, i1, i2, i3, v0, v1, v2, v3,
            si0, si1, si2, si3, sx0, sx1, sx2, sx3,
            sums_sh):
        cid = lax.axis_index("c")
        sid = lax.axis_index("s")
        wid = cid * 16 + sid
        idx_v = [i0, i1, i2, i3]
        x_v = [v0, v1, v2, v3]
        sem_i = [si0, si1, si2, si3]
        sem_x = [sx0, sx1, sx2, sx3]

        def start(j):
            base = (wid + N_WORKERS * j) * CHUNK
            pltpu.async_copy(b_hbm.at[pl.ds(base, CHUNK)], idx_v[j], sem_i[j])
            pltpu.async_copy(x_hbm.at[pl.ds(base, CHUNK)], x_v[j], sem_x[j])

        start(0)
        start(1)
        start(2)

        @pl.when(wid < FULL_WORKERS)
        def _():
            start(3)

        pltpu.sync_copy(z_hbm, sums_sh.at[pl.ds(sid * 16, 16)])
        plsc.subcore_barrier()

        def fire(j):
            base = (wid + N_WORKERS * j) * CHUNK
            pltpu.make_async_copy(
                b_hbm.at[pl.ds(base, CHUNK)], idx_v[j], sem_i[j]).wait()
            pltpu.make_async_copy(
                x_hbm.at[pl.ds(base, CHUNK)], x_v[j], sem_x[j]).wait()
            pltpu.async_copy(x_v[j], sums_sh.at[idx_v[j]], sem_x[j], add=True)

        def drain(j):
            pltpu.make_async_copy(
                x_v[j], sums_sh.at[idx_v[j]], sem_x[j]).wait()

        fire(0)
        fire(1)
        fire(2)

        @pl.when(wid < FULL_WORKERS)
        def _():
            fire(3)

        drain(0)
        drain(1)
        drain(2)

        @pl.when(wid < FULL_WORKERS)
        def _():
            drain(3)

        plsc.subcore_barrier()
        row = cid * NUM_GRAPHS + sid * 16
        pltpu.sync_copy(sums_sh.at[pl.ds(sid * 16, 16)],
                        sums_out.at[pl.ds(row, 16)])

    return seg(x, batch_i32, zeros_c)


def _mlp_body(sums_ref, b_ref, u_ref, w1_ref, b1_ref, gamma_ref,
              beta_ref, w2_ref, b2_ref, out_ref):
    s = sums_ref[...]
    total = s[:NUM_GRAPHS] + s[NUM_GRAPHS:]
    b = b_ref[...]
    segcol = lax.broadcasted_iota(jnp.int32, (NUM_GRAPHS, 1), 0)
    onehot = (b == segcol).astype(jnp.float32)
    cnt = jnp.sum(onehot, axis=1, keepdims=True)
    mean = total / jnp.maximum(cnt, 1.0)

    w1 = w1_ref[...]
    h = (jnp.dot(u_ref[...], w1[:GLOBAL_DIM], preferred_element_type=jnp.float32)
         + jnp.dot(mean, w1[GLOBAL_DIM:], preferred_element_type=jnp.float32)
         + b1_ref[...])
    mu = jnp.mean(h, axis=0, keepdims=True)
    var = jnp.mean((h - mu) ** 2, axis=0, keepdims=True)
    hn = (h - mu) * lax.rsqrt(var + 1e-5) * gamma_ref[...] + beta_ref[...]
    hn = jnp.maximum(hn, 0.0)
    out_ref[...] = (jnp.dot(hn, w2_ref[...], preferred_element_type=jnp.float32)
                    + b2_ref[...])


def kernel(x, edge_index, edge_attr, u, batch, W1, b1, gamma, beta, W2, b2):
    del edge_index, edge_attr
    batch_i32 = batch.astype(jnp.int32)
    zeros_c = jnp.zeros((16, D_FEAT), jnp.float32)
    batch_p = jnp.pad(batch_i32, (0, PAD_N - N_NODES),
                      constant_values=2**30).reshape(1, PAD_N)
    sums = _sc_segment_sums(x, batch_i32, zeros_c)
    out = pl.pallas_call(
        _mlp_body,
        out_shape=jax.ShapeDtypeStruct((NUM_GRAPHS, GLOBAL_DIM), jnp.float32),
    )(sums, batch_p, u, W1,
      b1.reshape(1, HIDDEN_DIM), gamma.reshape(1, HIDDEN_DIM),
      beta.reshape(1, HIDDEN_DIM), W2, b2.reshape(1, GLOBAL_DIM))
    return out


# speedup vs baseline: 1.0341x; 1.0341x over previous
import functools

import jax
import jax.numpy as jnp
from jax import lax
from jax.experimental import pallas as pl
from jax.experimental.pallas import tpu as pltpu
from jax.experimental.pallas import tpu_sc as plsc

N_NODES = 10000
D_FEAT = 128
NUM_GRAPHS = 256
GLOBAL_DIM = 64
HIDDEN_DIM = 256

N_WORKERS = 32
CHUNK = 312
TAIL = N_NODES - N_WORKERS * CHUNK
TAIL_BASE = N_WORKERS * CHUNK
PAD_N = 10240


def _sc_segment_sums(x, batch_i32, zeros_c):
    mesh = plsc.VectorSubcoreMesh(core_axis_name="c", subcore_axis_name="s")

    @functools.partial(
        pl.kernel,
        out_type=jax.ShapeDtypeStruct((2 * NUM_GRAPHS, D_FEAT), jnp.float32),
        mesh=mesh,
        scratch_types=(
            [pltpu.VMEM((CHUNK, D_FEAT), jnp.float32),
             pltpu.VMEM((CHUNK,), jnp.int32),
             pltpu.VMEM((TAIL, D_FEAT), jnp.float32),
             pltpu.VMEM((TAIL,), jnp.int32)]
            + [pltpu.SemaphoreType.DMA for _ in range(4)]
            + [pltpu.VMEM_SHARED((NUM_GRAPHS, D_FEAT), jnp.float32)]
        ),
    )
    def seg(x_hbm, b_hbm, z_hbm, sums_out,
            xv, iv, xt, it, sx, si, stx, sti,
            sums_sh):
        cid = lax.axis_index("c")
        sid = lax.axis_index("s")
        wid = cid * 16 + sid
        base = wid * CHUNK
        is_tail = wid == N_WORKERS - 1

        pltpu.async_copy(b_hbm.at[pl.ds(base, CHUNK)], iv, si)
        pltpu.async_copy(x_hbm.at[pl.ds(base, CHUNK)], xv, sx)

        @pl.when(is_tail)
        def _():
            pltpu.async_copy(b_hbm.at[pl.ds(TAIL_BASE, TAIL)], it, sti)
            pltpu.async_copy(x_hbm.at[pl.ds(TAIL_BASE, TAIL)], xt, stx)

        pltpu.sync_copy(z_hbm, sums_sh.at[pl.ds(sid * 16, 16)])
        plsc.subcore_barrier()

        pltpu.make_async_copy(b_hbm.at[pl.ds(base, CHUNK)], iv, si).wait()
        pltpu.make_async_copy(x_hbm.at[pl.ds(base, CHUNK)], xv, sx).wait()
        pltpu.async_copy(xv, sums_sh.at[iv], sx, add=True)

        @pl.when(is_tail)
        def _():
            pltpu.make_async_copy(
                b_hbm.at[pl.ds(TAIL_BASE, TAIL)], it, sti).wait()
            pltpu.make_async_copy(
                x_hbm.at[pl.ds(TAIL_BASE, TAIL)], xt, stx).wait()
            pltpu.async_copy(xt, sums_sh.at[it], stx, add=True)
            pltpu.make_async_copy(xt, sums_sh.at[it], stx).wait()

        pltpu.make_async_copy(xv, sums_sh.at[iv], sx).wait()

        plsc.subcore_barrier()
        row = cid * NUM_GRAPHS + sid * 16
        pltpu.sync_copy(sums_sh.at[pl.ds(sid * 16, 16)],
                        sums_out.at[pl.ds(row, 16)])

    return seg(x, batch_i32, zeros_c)


def _counts_body(b_ref, out_ref):
    b = b_ref[...]
    segcol = lax.broadcasted_iota(jnp.int32, (NUM_GRAPHS, 1), 0)
    onehot = (b == segcol).astype(jnp.float32)
    out_ref[...] = jnp.sum(onehot, axis=1, keepdims=True)


def _mlp_body(sums_ref, cnts_ref, u_ref, w1_ref, b1_ref, gamma_ref,
              beta_ref, w2_ref, b2_ref, out_ref):
    s = sums_ref[...]
    total = s[:NUM_GRAPHS] + s[NUM_GRAPHS:]
    cnt = cnts_ref[...]
    mean = total / jnp.maximum(cnt, 1.0)

    w1 = w1_ref[...]
    h = (jnp.dot(u_ref[...], w1[:GLOBAL_DIM], preferred_element_type=jnp.float32)
         + jnp.dot(mean, w1[GLOBAL_DIM:], preferred_element_type=jnp.float32)
         + b1_ref[...])
    mu = jnp.mean(h, axis=0, keepdims=True)
    var = jnp.mean((h - mu) ** 2, axis=0, keepdims=True)
    hn = (h - mu) * lax.rsqrt(var + 1e-5) * gamma_ref[...] + beta_ref[...]
    hn = jnp.maximum(hn, 0.0)
    out_ref[...] = (jnp.dot(hn, w2_ref[...], preferred_element_type=jnp.float32)
                    + b2_ref[...])


def kernel(x, edge_index, edge_attr, u, batch, W1, b1, gamma, beta, W2, b2):
    del edge_index, edge_attr
    batch_i32 = batch.astype(jnp.int32)
    zeros_c = jnp.zeros((16, D_FEAT), jnp.float32)
    batch_p = jnp.pad(batch_i32, (0, PAD_N - N_NODES),
                      constant_values=2**30).reshape(1, PAD_N)
    cnts = pl.pallas_call(
        _counts_body,
        out_shape=jax.ShapeDtypeStruct((NUM_GRAPHS, 1), jnp.float32),
    )(batch_p)
    sums = _sc_segment_sums(x, batch_i32, zeros_c)
    out = pl.pallas_call(
        _mlp_body,
        out_shape=jax.ShapeDtypeStruct((NUM_GRAPHS, GLOBAL_DIM), jnp.float32),
    )(sums, cnts, u, W1,
      b1.reshape(1, HIDDEN_DIM), gamma.reshape(1, HIDDEN_DIM),
      beta.reshape(1, HIDDEN_DIM), W2, b2.reshape(1, GLOBAL_DIM))
    return out
